# baseline (speedup 1.0000x reference)
import jax
import jax.numpy as jnp
from jax.experimental import pallas as pl
from jax.experimental.pallas import tpu as pltpu

_N = 14


def _make_kernel(t):
    def _gcn_fused_kernel(x_ref, a_ref, w_ref, b_ref, fw_ref, o_ref, acc_ref):
        j = pl.program_id(1)
        n = a_ref.shape[0]

        part = jnp.dot(x_ref[...], w_ref[...],
                       preferred_element_type=jnp.float32)

        @pl.when(j == 0)
        def _init():
            acc_ref[...] = part

        @pl.when(j > 0)
        def _acc():
            acc_ref[...] += part

        @pl.when(j == t - 1)
        def _finish():
            a = a_ref[...]
            row = jax.lax.broadcasted_iota(jnp.int32, (n, n), 0)
            col = jax.lax.broadcasted_iota(jnp.int32, (n, n), 1)
            eye = (row == col).astype(jnp.float32)
            a = jnp.where(a_ref[0:1, 0:1] == 0.0, a + eye, a)

            axw = jnp.dot(a, acc_ref[...], preferred_element_type=jnp.float32)
            gc = jnp.maximum(axw, 0.0) + b_ref[...]

            o_ref[...] = jnp.sum(fw_ref[...] * gc[None, :, :],
                                 axis=(1, 2)).reshape(1, 1, n)

    return _gcn_fused_kernel


def kernel(x, adj, gc_weight, gc_bias, fc_weight, fc_bias):
    n = _N
    x2d = x.reshape(n, -1).astype(jnp.float32)
    f_dim = x2d.shape[1]
    p_dim = gc_weight.shape[1]
    w = gc_weight.astype(jnp.float32)
    a = adj.astype(jnp.float32)
    b2 = gc_bias.reshape(1, p_dim).astype(jnp.float32)
    fw3 = fc_weight.reshape(n, n, p_dim).astype(jnp.float32)

    ncores = 2 if p_dim % 256 == 0 else 1
    ph = p_dim // ncores
    ft = f_dim // 4 if f_dim % (4 * 128) == 0 else f_dim
    t = f_dim // ft

    parts = pl.pallas_call(
        _make_kernel(t),
        grid=(ncores, t),
        in_specs=[
            pl.BlockSpec((n, ft), lambda c, j: (0, j)),
            pl.BlockSpec((n, n), lambda c, j: (0, 0)),
            pl.BlockSpec((ft, ph), lambda c, j: (j, c)),
            pl.BlockSpec((1, ph), lambda c, j: (0, c)),
            pl.BlockSpec((n, n, ph), lambda c, j: (0, 0, c)),
        ],
        out_specs=pl.BlockSpec((1, 1, n), lambda c, j: (c, 0, 0)),
        out_shape=jax.ShapeDtypeStruct((ncores, 1, n), jnp.float32),
        scratch_shapes=[pltpu.VMEM((n, ph), jnp.float32)],
        compiler_params=pltpu.CompilerParams(
            dimension_semantics=("parallel", "arbitrary")),
    )(x2d, a, w, b2, fw3)

    out = parts.reshape(ncores, n).sum(axis=0, keepdims=True)
    return out + fc_bias.reshape(1, n).astype(jnp.float32)
